# baseline (device time: 18042 ns/iter reference)
import jax
import jax.numpy as jnp
from jax import lax
from jax.experimental import pallas as pl
from jax.experimental.pallas import tpu as pltpu

NB = 4


def kernel(dy, W):
    M, K = dy.shape
    D = W.shape[0]
    QR = M // 4
    PR = 2 * QR
    CB = D // NB
    HB = D // 2

    def body(dy_ref, w_ref, out_ref, dy_buf, w_buf, rx_own, rx_diag,
             in_sems, sx, rxs, sy, ry, sz, rz):
        x = lax.axis_index("x")
        y = lax.axis_index("y")
        z = lax.axis_index("z")
        e = y ^ z
        own = (1 - e) * y + e * (2 + z)
        y_own = e * (1 - y) + (1 - e) * (2 + z)
        z_own = e * y + (1 - e) * (3 - z)
        o_row = own * QR
        pair_row = e * PR
        own_off = ((1 - e) * y + e * z) * QR
        diag_off = QR - own_off
        d_row = pair_row + diag_off

        xp = (1 - x, y, z)
        yp = (x, 1 - y, z)
        zp = (x, y, 1 - z)

        dy_cp = pltpu.make_async_copy(
            dy_ref.at[pl.ds(pair_row, PR)], dy_buf, in_sems.at[NB])
        dy_cp.start()
        w_cps = []
        for j in range(NB):
            cp = pltpu.make_async_copy(
                w_ref.at[pl.ds(j * CB, CB)], w_buf.at[pl.ds(j * CB, CB)],
                in_sems.at[j])
            cp.start()
            w_cps.append(cp)

        barrier = pltpu.get_barrier_semaphore()
        for dev in [xp, yp, zp]:
            pl.semaphore_signal(
                barrier, inc=1, device_id=dev,
                device_id_type=pl.DeviceIdType.MESH,
            )
        pl.semaphore_wait(barrier, 3)
        dy_cp.wait()

        def gemm(dy_rows_off, rows, wj_off, cols):
            return lax.dot_general(
                dy_buf[pl.ds(dy_rows_off, rows), :],
                w_buf[pl.ds(wj_off, cols), :],
                (((1,), (1,)), ((), ())),
                preferred_element_type=jnp.float32,
            )

        x_rdmas = []
        for j in range(NB):
            w_cps[j].wait()
            out_ref[pl.ds(o_row, QR), pl.ds(j * CB, CB)] = gemm(
                own_off, QR, j * CB, CB)
            r = pltpu.make_async_remote_copy(
                src_ref=out_ref.at[pl.ds(o_row, QR), pl.ds(j * CB, CB)],
                dst_ref=rx_own.at[:, pl.ds(j * CB, CB)],
                send_sem=sx.at[j],
                recv_sem=rxs.at[j],
                device_id=xp,
                device_id_type=pl.DeviceIdType.MESH,
            )
            r.start()
            x_rdmas.append(r)

        xd_rdmas = []
        for c in range(2):
            out_ref[pl.ds(d_row, QR), pl.ds(c * HB, HB)] = gemm(
                diag_off, QR, c * HB, HB)
            r = pltpu.make_async_remote_copy(
                src_ref=out_ref.at[pl.ds(d_row, QR), pl.ds(c * HB, HB)],
                dst_ref=rx_diag.at[:, pl.ds(c * HB, HB)],
                send_sem=sx.at[NB + c],
                recv_sem=rxs.at[NB + c],
                device_id=xp,
                device_id_type=pl.DeviceIdType.MESH,
            )
            r.start()
            xd_rdmas.append(r)

        yz_rdmas = []
        for c in range(2):
            for j in (2 * c, 2 * c + 1):
                x_rdmas[j].wait()
                out_ref[pl.ds(o_row, QR), pl.ds(j * CB, CB)] = (
                    out_ref[pl.ds(o_row, QR), pl.ds(j * CB, CB)]
                    + rx_own[:, pl.ds(j * CB, CB)]
                )
            for sem_s, sem_r, dev in ((sy, ry, yp), (sz, rz, zp)):
                r = pltpu.make_async_remote_copy(
                    src_ref=out_ref.at[pl.ds(o_row, QR), pl.ds(c * HB, HB)],
                    dst_ref=out_ref.at[pl.ds(o_row, QR), pl.ds(c * HB, HB)],
                    send_sem=sem_s.at[c],
                    recv_sem=sem_r.at[c],
                    device_id=dev,
                    device_id_type=pl.DeviceIdType.MESH,
                )
                r.start()
                yz_rdmas.append(r)

        for c in range(2):
            xd_rdmas[c].wait()
            out_ref[pl.ds(d_row, QR), pl.ds(c * HB, HB)] = (
                out_ref[pl.ds(d_row, QR), pl.ds(c * HB, HB)]
                + rx_diag[:, pl.ds(c * HB, HB)]
            )

        for sem_s, sem_r, dev, src_q in (
            (sy, ry, yp, y_own),
            (sz, rz, zp, z_own),
        ):
            for c in range(2):
                r = pltpu.make_async_remote_copy(
                    src_ref=out_ref.at[pl.ds(src_q * QR, QR), pl.ds(c * HB, HB)],
                    dst_ref=out_ref.at[pl.ds(src_q * QR, QR), pl.ds(c * HB, HB)],
                    send_sem=sem_s.at[c],
                    recv_sem=sem_r.at[c],
                    device_id=dev,
                    device_id_type=pl.DeviceIdType.MESH,
                )
                r.wait_recv()

        for r in yz_rdmas:
            r.wait_send()

    return pl.pallas_call(
        body,
        out_shape=jax.ShapeDtypeStruct((M, D), jnp.float32),
        in_specs=[
            pl.BlockSpec(memory_space=pl.ANY),
            pl.BlockSpec(memory_space=pl.ANY),
        ],
        out_specs=pl.BlockSpec(memory_space=pltpu.VMEM),
        scratch_shapes=[
            pltpu.VMEM((PR, K), jnp.float32),
            pltpu.VMEM((D, K), jnp.float32),
            pltpu.VMEM((QR, D), jnp.float32),
            pltpu.VMEM((QR, D), jnp.float32),
            pltpu.SemaphoreType.DMA((NB + 1,)),
            pltpu.SemaphoreType.DMA((NB + 2,)),
            pltpu.SemaphoreType.DMA((NB + 2,)),
            pltpu.SemaphoreType.DMA((2,)),
            pltpu.SemaphoreType.DMA((2,)),
            pltpu.SemaphoreType.DMA((2,)),
            pltpu.SemaphoreType.DMA((2,)),
        ],
        compiler_params=pltpu.CompilerParams(collective_id=0),
    )(dy, W)
